# baseline (speedup 1.0000x reference)
import functools

import jax
import jax.numpy as jnp
from jax import lax
from jax.experimental import pallas as pl
from jax.experimental.pallas import tpu as pltpu
from jax.experimental.pallas import tpu_sc as plsc

B = 32
N = 100000
A = 8
NB = 5120
NBLK = (N + NB - 1) // NB
NPAD = NBLK * NB
EXPLOIT = 0.9
NEG_INF = float("-inf")
BIG_I32 = 2**30


def _tc_body(qm_ref, qs_ref, eps_ref, lp_ref, m_out_ref, idx_out_ref,
             u_sc, m_sc, s_sc, i_sc):
    ph = pl.program_id(0)
    j = pl.program_id(1)
    off = pl.multiple_of(j * NB, NB)

    @pl.when(ph == 0)
    def _phase0():
        qs = qs_ref[...]
        u = EXPLOIT * (qm_ref[...] + qs * eps_ref[...]) + (1.0 - EXPLOIT) * qs
        u_sc[:, pl.ds(off, NB)] = u
        lane = lax.broadcasted_iota(jnp.int32, (B, NB), 1) + off
        valid = lane < N
        um = jnp.where(valid, u, NEG_INF)
        bm = jnp.max(um, axis=1, keepdims=True)
        bidx = jnp.min(jnp.where(um == bm, lane, BIG_I32), axis=1,
                       keepdims=True)

        @pl.when(j == 0)
        def _():
            m_sc[...] = bm
            s_sc[...] = jnp.sum(jnp.where(valid, jnp.exp(u - bm), 0.0),
                                axis=1, keepdims=True)
            i_sc[...] = bidx

        @pl.when(j > 0)
        def _():
            m_old = m_sc[...]
            m_new = jnp.maximum(m_old, bm)
            s_sc[...] = (s_sc[...] * jnp.exp(m_old - m_new)
                         + jnp.sum(jnp.where(valid, jnp.exp(u - m_new), 0.0),
                                   axis=1, keepdims=True))
            i_sc[...] = jnp.where(bm > m_old, bidx, i_sc[...])
            m_sc[...] = m_new

        @pl.when(j == NBLK - 1)
        def _():
            m_out_ref[...] = m_sc[...]
            idx_out_ref[...] = jnp.broadcast_to(i_sc[...], (B, 128))

    @pl.when(ph == 1)
    def _phase1():
        lse = m_sc[...] + jnp.log(s_sc[...])
        lp_ref[...] = u_sc[:, pl.ds(off, NB)] - lse


def _tc_call(q_mean, q_std, eps, interpret=False):
    in_spec = pl.BlockSpec((B, NB), lambda ph, j: (0, jnp.where(ph == 0, j, 0)))
    return pl.pallas_call(
        _tc_body,
        grid=(2, NBLK),
        in_specs=[in_spec, in_spec, in_spec],
        out_specs=[
            pl.BlockSpec((B, NB), lambda ph, j: (0, jnp.where(ph == 0, 0, j))),
            pl.BlockSpec((B, 1), lambda ph, j: (0, 0)),
            pl.BlockSpec((B, 128), lambda ph, j: (0, 0)),
        ],
        out_shape=[
            jax.ShapeDtypeStruct((B, N), jnp.float32),
            jax.ShapeDtypeStruct((B, 1), jnp.float32),
            jax.ShapeDtypeStruct((B, 128), jnp.int32),
        ],
        scratch_shapes=[
            pltpu.VMEM((B, NPAD), jnp.float32),
            pltpu.VMEM((B, 1), jnp.float32),
            pltpu.VMEM((B, 1), jnp.float32),
            pltpu.VMEM((B, 1), jnp.int32),
        ],
        compiler_params=pltpu.CompilerParams(
            dimension_semantics=("arbitrary", "arbitrary")),
        interpret=interpret,
    )(q_mean, q_std, eps)


@functools.cache
def _sc_gather_fn():
    mesh = plsc.VectorSubcoreMesh(core_axis_name="c", subcore_axis_name="s")

    @functools.partial(
        pl.kernel,
        out_type=jax.ShapeDtypeStruct((B, 128), jnp.float32),
        mesh=mesh,
        scratch_types=[
            pltpu.VMEM((128,), jnp.int32),
            pltpu.VMEM((1,), jnp.int32),
            pltpu.VMEM((1, 128), jnp.float32),
            pltpu.VMEM((128,), jnp.float32),
            pltpu.SemaphoreType.DMA,
        ],
        compiler_params=pltpu.CompilerParams(needs_layout_passes=False),
    )
    def _sc_gather(table_hbm, idx_hbm, out_hbm, idx_v, rowidx_v, row_v,
                   out_v, sem):
        wid = lax.axis_index("s") * 2 + lax.axis_index("c")

        @pl.when(wid < B)
        def _():
            pltpu.sync_copy(idx_hbm.at[wid], idx_v)
            vec = idx_v[pl.ds(0, 16)]
            lanes = lax.iota(jnp.int32, 16)
            row128 = wid * (N * A // 128) + lax.shift_right_logical(vec, 4)
            plsc.store_scatter(rowidx_v, [lanes], row128, mask=lanes < 1)
            pltpu.async_copy(table_hbm.at[rowidx_v], row_v, sem).wait()
            col = (vec & 15) * A + lanes
            vals = plsc.load_gather(row_v, [lanes * 0, col])
            plsc.store_scatter(out_v, [lanes], vals, mask=lanes < A)
            pltpu.sync_copy(out_v, out_hbm.at[wid])

    return _sc_gather


def kernel(q_mean, q_std, eps, action):
    log_probs, m, idx128 = _tc_call(q_mean, q_std, eps)
    table = action.reshape(B * N * A // 128, 128)
    best_action = _sc_gather_fn()(table, idx128)[:, :A]
    return log_probs, best_action, m.reshape(B)


# speedup vs baseline: 31.0553x; 31.0553x over previous
import functools

import jax
import jax.numpy as jnp
from jax import lax
from jax.experimental import pallas as pl
from jax.experimental.pallas import tpu as pltpu
from jax.experimental.pallas import tpu_sc as plsc

B = 32
N = 100000
A = 8
NB = 5120
NBLK = (N + NB - 1) // NB
NPAD = NBLK * NB
EXPLOIT = 0.9
NEG_INF = float("-inf")
BIG_I32 = 2**30


def _tc_body(qm_ref, qs_ref, eps_ref, lp_ref, m_out_ref, idx_out_ref,
             u_sc, m_sc, s_sc, i_sc):
    ph = pl.program_id(0)
    j = pl.program_id(1)
    off = pl.multiple_of(j * NB, NB)

    @pl.when(ph == 0)
    def _phase0():
        qs = qs_ref[...]
        u = EXPLOIT * (qm_ref[...] + qs * eps_ref[...]) + (1.0 - EXPLOIT) * qs
        u_sc[:, pl.ds(off, NB)] = u
        lane = lax.broadcasted_iota(jnp.int32, (B, NB), 1) + off
        valid = lane < N
        um = jnp.where(valid, u, NEG_INF)
        bm = jnp.max(um, axis=1, keepdims=True)
        bidx = jnp.min(jnp.where(um == bm, lane, BIG_I32), axis=1,
                       keepdims=True)

        @pl.when(j == 0)
        def _():
            m_sc[...] = bm
            s_sc[...] = jnp.sum(jnp.where(valid, jnp.exp(u - bm), 0.0),
                                axis=1, keepdims=True)
            i_sc[...] = bidx

        @pl.when(j > 0)
        def _():
            m_old = m_sc[...]
            m_new = jnp.maximum(m_old, bm)
            s_sc[...] = (s_sc[...] * jnp.exp(m_old - m_new)
                         + jnp.sum(jnp.where(valid, jnp.exp(u - m_new), 0.0),
                                   axis=1, keepdims=True))
            i_sc[...] = jnp.where(bm > m_old, bidx, i_sc[...])
            m_sc[...] = m_new

        @pl.when(j == NBLK - 1)
        def _():
            m_out_ref[...] = m_sc[...]
            idx_out_ref[...] = i_sc[...]

    @pl.when(ph == 1)
    def _phase1():
        lse = m_sc[...] + jnp.log(s_sc[...])
        lp_ref[...] = u_sc[:, pl.ds(off, NB)] - lse


def _tc_call(q_mean, q_std, eps, interpret=False):
    in_spec = pl.BlockSpec((B, NB), lambda ph, j: (0, jnp.where(ph == 0, j, 0)))
    return pl.pallas_call(
        _tc_body,
        grid=(2, NBLK),
        in_specs=[in_spec, in_spec, in_spec],
        out_specs=[
            pl.BlockSpec((B, NB), lambda ph, j: (0, jnp.where(ph == 0, 0, j))),
            pl.BlockSpec((B, 1), lambda ph, j: (0, 0)),
            pl.BlockSpec((B, 1), lambda ph, j: (0, 0)),
        ],
        out_shape=[
            jax.ShapeDtypeStruct((B, N), jnp.float32),
            jax.ShapeDtypeStruct((B, 1), jnp.float32),
            jax.ShapeDtypeStruct((B, 1), jnp.int32),
        ],
        scratch_shapes=[
            pltpu.VMEM((B, NPAD), jnp.float32),
            pltpu.VMEM((B, 1), jnp.float32),
            pltpu.VMEM((B, 1), jnp.float32),
            pltpu.VMEM((B, 1), jnp.int32),
        ],
        compiler_params=pltpu.CompilerParams(
            dimension_semantics=("arbitrary", "arbitrary")),
        interpret=interpret,
    )(q_mean, q_std, eps)


def _gather_body(at_ref, idx_smem, idxv_ref, ba_ref, win, sem):
    for b in range(B):
        idxb = idx_smem[b]
        col0 = jnp.minimum((idxb // 128) * 128, N - 128)
        pltpu.make_async_copy(
            at_ref.at[b, :, pl.ds(pl.multiple_of(col0, 128), 128)],
            win.at[b], sem).start()
    for b in range(B):
        idxb = idx_smem[b]
        col0 = jnp.minimum((idxb // 128) * 128, N - 128)
        pltpu.make_async_copy(
            at_ref.at[b, :, pl.ds(pl.multiple_of(col0, 128), 128)],
            win.at[b], sem).wait()
    idxv = idxv_ref[...].reshape(B, 1, 1)
    col0v = jnp.minimum((idxv // 128) * 128, N - 128)
    cw = idxv - col0v
    lane = lax.broadcasted_iota(jnp.int32, (B, A, 128), 2)
    ba_ref[...] = jnp.sum(jnp.where(lane == cw, win[...], 0.0), axis=2)


def _gather_call(at, idx_s, idx_v):
    return pl.pallas_call(
        _gather_body,
        in_specs=[
            pl.BlockSpec(memory_space=pl.ANY),
            pl.BlockSpec(memory_space=pltpu.SMEM),
            pl.BlockSpec((B, 1), lambda: (0, 0)),
        ],
        out_specs=pl.BlockSpec((B, A), lambda: (0, 0)),
        out_shape=jax.ShapeDtypeStruct((B, A), jnp.float32),
        scratch_shapes=[
            pltpu.VMEM((B, A, 128), jnp.float32),
            pltpu.SemaphoreType.DMA,
        ],
    )(at, idx_s, idx_v)


def kernel(q_mean, q_std, eps, action):
    log_probs, m, idx1 = _tc_call(q_mean, q_std, eps)
    at = action.transpose(0, 2, 1)
    best_action = _gather_call(at, idx1.reshape(B), idx1)
    return log_probs, best_action, m.reshape(B)


# speedup vs baseline: 35.3691x; 1.1389x over previous
import jax
import jax.numpy as jnp
from jax import lax
from jax.experimental import pallas as pl
from jax.experimental.pallas import tpu as pltpu

B = 32
N = 100000
A = 8
NB = 6400
NBLK = (N + NB - 1) // NB
NPAD = NBLK * NB
EXPLOIT = 0.9
NEG_INF = float("-inf")
BIG_I32 = 2**30


def _tc_body(qm_ref, qs_ref, eps_ref, at_ref,
             lp_ref, m_out_ref, ba_ref,
             u_sc, m_sc, s_sc, i_sc, idx_smem, win, sem, gsem):
    ph = pl.program_id(0)
    j = pl.program_id(1)
    off = pl.multiple_of(j * NB, NB)

    def _update(u, um, lane):
        bm = jnp.max(um, axis=1, keepdims=True)
        bidx = jnp.min(jnp.where(um == bm, lane, BIG_I32), axis=1,
                       keepdims=True)

        @pl.when(j == 0)
        def _():
            m_sc[...] = bm
            s_sc[...] = jnp.sum(jnp.exp(um - bm), axis=1, keepdims=True)
            i_sc[...] = bidx

        @pl.when(j > 0)
        def _():
            m_old = m_sc[...]
            m_new = jnp.maximum(m_old, bm)
            s_sc[...] = (s_sc[...] * jnp.exp(m_old - m_new)
                         + jnp.sum(jnp.exp(um - m_new), axis=1,
                                   keepdims=True))
            i_sc[...] = jnp.where(bm > m_old, bidx, i_sc[...])
            m_sc[...] = m_new

    @pl.when(ph == 0)
    def _phase0():
        qs = qs_ref[...]
        u = EXPLOIT * (qm_ref[...] + qs * eps_ref[...]) + (1.0 - EXPLOIT) * qs
        u_sc[:, pl.ds(off, NB)] = u
        lane = lax.broadcasted_iota(jnp.int32, (B, NB), 1) + off

        @pl.when(j < NBLK - 1)
        def _():
            _update(u, u, lane)

        @pl.when(j == NBLK - 1)
        def _():
            um = jnp.where(lane < N, u, NEG_INF)
            _update(u, um, lane)
            m_out_ref[...] = m_sc[...]

    @pl.when(ph == 1)
    def _phase1():
        lse = m_sc[...] + jnp.log(s_sc[...])
        lp_ref[...] = u_sc[:, pl.ds(off, NB)] - lse

        @pl.when(j == 0)
        def _gather():
            pltpu.make_async_copy(i_sc, idx_smem, gsem).start()
            pltpu.make_async_copy(i_sc, idx_smem, gsem).wait()
            for b in range(B):
                col0 = jnp.minimum((idx_smem[b, 0] // 128) * 128, N - 128)
                pltpu.make_async_copy(
                    at_ref.at[b, :, pl.ds(pl.multiple_of(col0, 128), 128)],
                    win.at[b], sem).start()
            for b in range(B):
                col0 = jnp.minimum((idx_smem[b, 0] // 128) * 128, N - 128)
                pltpu.make_async_copy(
                    at_ref.at[b, :, pl.ds(pl.multiple_of(col0, 128), 128)],
                    win.at[b], sem).wait()
            idxv = i_sc[...].reshape(B, 1, 1)
            cw = idxv - jnp.minimum((idxv // 128) * 128, N - 128)
            lane3 = lax.broadcasted_iota(jnp.int32, (B, A, 128), 2)
            ba_ref[...] = jnp.sum(
                jnp.where(lane3 == cw, win[...], 0.0), axis=2)


def _tc_call(q_mean, q_std, eps, at):
    in_spec = pl.BlockSpec((B, NB), lambda ph, j: (0, jnp.where(ph == 0, j, 0)))
    return pl.pallas_call(
        _tc_body,
        grid=(2, NBLK),
        in_specs=[
            in_spec, in_spec, in_spec,
            pl.BlockSpec(memory_space=pl.ANY),
        ],
        out_specs=[
            pl.BlockSpec((B, NB), lambda ph, j: (0, jnp.where(ph == 0, 0, j))),
            pl.BlockSpec((B, 1), lambda ph, j: (0, 0)),
            pl.BlockSpec((B, A), lambda ph, j: (0, 0)),
        ],
        out_shape=[
            jax.ShapeDtypeStruct((B, N), jnp.float32),
            jax.ShapeDtypeStruct((B, 1), jnp.float32),
            jax.ShapeDtypeStruct((B, A), jnp.float32),
        ],
        scratch_shapes=[
            pltpu.VMEM((B, NPAD), jnp.float32),
            pltpu.VMEM((B, 1), jnp.float32),
            pltpu.VMEM((B, 1), jnp.float32),
            pltpu.VMEM((B, 1), jnp.int32),
            pltpu.SMEM((B, 1), jnp.int32),
            pltpu.VMEM((B, A, 128), jnp.float32),
            pltpu.SemaphoreType.DMA,
            pltpu.SemaphoreType.DMA,
        ],
        compiler_params=pltpu.CompilerParams(
            dimension_semantics=("arbitrary", "arbitrary")),
    )(q_mean, q_std, eps, at)


def kernel(q_mean, q_std, eps, action):
    at = action.transpose(0, 2, 1)
    log_probs, m, best_action = _tc_call(q_mean, q_std, eps, at)
    return log_probs, best_action, m.reshape(B)


# speedup vs baseline: 36.7691x; 1.0396x over previous
import jax
import jax.numpy as jnp
from jax import lax
from jax.experimental import pallas as pl
from jax.experimental.pallas import tpu as pltpu

B = 32
N = 100000
A = 8
NB = 6400
NBLK = (N + NB - 1) // NB
NPAD = NBLK * NB
EXPLOIT = 0.9
NEG_INF = float("-inf")
BIG_I32 = 2**30


def _tc_body(qm_ref, qs_ref, eps_ref, at_ref,
             lp_ref, m_out_ref, ba_ref,
             u_sc, m_sc, s_sc, i_sc, idx_smem, win, sem, gsem):
    ph = pl.program_id(0)
    j = pl.program_id(1)
    off = pl.multiple_of(j * NB, NB)

    def _update(u, um, lane):
        bm = jnp.max(um, axis=1, keepdims=True)
        bidx = jnp.min(jnp.where(um == bm, lane, BIG_I32), axis=1,
                       keepdims=True)

        @pl.when(j == 0)
        def _():
            m_sc[...] = bm
            s_sc[...] = jnp.sum(jnp.exp(um - bm), axis=1, keepdims=True)
            i_sc[...] = bidx

        @pl.when(j > 0)
        def _():
            m_old = m_sc[...]
            m_new = jnp.maximum(m_old, bm)
            s_sc[...] = (s_sc[...] * jnp.exp(m_old - m_new)
                         + jnp.sum(jnp.exp(um - m_new), axis=1,
                                   keepdims=True))
            i_sc[...] = jnp.where(bm > m_old, bidx, i_sc[...])
            m_sc[...] = m_new

    @pl.when(ph == 0)
    def _phase0():
        qs = qs_ref[...]
        u = EXPLOIT * (qm_ref[...] + qs * eps_ref[...]) + (1.0 - EXPLOIT) * qs
        u_sc[:, pl.ds(off, NB)] = u
        lane = lax.broadcasted_iota(jnp.int32, (B, NB), 1) + off

        @pl.when(j < NBLK - 1)
        def _():
            _update(u, u, lane)

        @pl.when(j == NBLK - 1)
        def _():
            um = jnp.where(lane < N, u, NEG_INF)
            _update(u, um, lane)
            m_out_ref[...] = m_sc[...]
            pltpu.make_async_copy(i_sc, idx_smem, gsem).start()
            pltpu.make_async_copy(i_sc, idx_smem, gsem).wait()
            for b in range(B):
                col0 = jnp.minimum((idx_smem[b, 0] // 128) * 128, N - 128)
                pltpu.make_async_copy(
                    at_ref.at[b, :, pl.ds(pl.multiple_of(col0, 128), 128)],
                    win.at[b], sem).start()

    @pl.when(ph == 1)
    def _phase1():
        lse = m_sc[...] + jnp.log(s_sc[...])
        lp_ref[...] = u_sc[:, pl.ds(off, NB)] - lse

        @pl.when(j == NBLK - 1)
        def _gather():
            for b in range(B):
                col0 = jnp.minimum((idx_smem[b, 0] // 128) * 128, N - 128)
                pltpu.make_async_copy(
                    at_ref.at[b, :, pl.ds(pl.multiple_of(col0, 128), 128)],
                    win.at[b], sem).wait()
            idxv = i_sc[...].reshape(B, 1, 1)
            cw = idxv - jnp.minimum((idxv // 128) * 128, N - 128)
            lane3 = lax.broadcasted_iota(jnp.int32, (B, A, 128), 2)
            ba_ref[...] = jnp.sum(
                jnp.where(lane3 == cw, win[...], 0.0), axis=2)


def _tc_call(q_mean, q_std, eps, at):
    in_spec = pl.BlockSpec((B, NB), lambda ph, j: (0, jnp.where(ph == 0, j, 0)))
    return pl.pallas_call(
        _tc_body,
        grid=(2, NBLK),
        in_specs=[
            in_spec, in_spec, in_spec,
            pl.BlockSpec(memory_space=pl.ANY),
        ],
        out_specs=[
            pl.BlockSpec((B, NB), lambda ph, j: (0, jnp.where(ph == 0, 0, j))),
            pl.BlockSpec((B, 1), lambda ph, j: (0, 0)),
            pl.BlockSpec((B, A), lambda ph, j: (0, 0)),
        ],
        out_shape=[
            jax.ShapeDtypeStruct((B, N), jnp.float32),
            jax.ShapeDtypeStruct((B, 1), jnp.float32),
            jax.ShapeDtypeStruct((B, A), jnp.float32),
        ],
        scratch_shapes=[
            pltpu.VMEM((B, NPAD), jnp.float32),
            pltpu.VMEM((B, 1), jnp.float32),
            pltpu.VMEM((B, 1), jnp.float32),
            pltpu.VMEM((B, 1), jnp.int32),
            pltpu.SMEM((B, 1), jnp.int32),
            pltpu.VMEM((B, A, 128), jnp.float32),
            pltpu.SemaphoreType.DMA,
            pltpu.SemaphoreType.DMA,
        ],
        compiler_params=pltpu.CompilerParams(
            dimension_semantics=("arbitrary", "arbitrary")),
    )(q_mean, q_std, eps, at)


def kernel(q_mean, q_std, eps, action):
    at = action.transpose(0, 2, 1)
    log_probs, m, best_action = _tc_call(q_mean, q_std, eps, at)
    return log_probs, best_action, m.reshape(B)


# speedup vs baseline: 48.2613x; 1.3126x over previous
import jax
import jax.numpy as jnp
from jax import lax
from jax.experimental import pallas as pl
from jax.experimental.pallas import tpu as pltpu

B = 32
N = 100000
A = 8
NB = 12800
NBLK = (N + NB - 1) // NB
NPAD = NBLK * NB
EXPLOIT = 0.9
NEG_INF = float("-inf")
BIG_I32 = 2**30


def _tc_body(qm_ref, qs_ref, eps_ref, at_ref,
             lp_ref, m_out_ref, ba_ref,
             u_sc, m_sc, s_sc, i_sc, idx_smem, win, sem, gsem):
    ph = pl.program_id(0)
    j = pl.program_id(1)
    off = pl.multiple_of(j * NB, NB)

    def _update(u, um, lane):
        bm = jnp.max(um, axis=1, keepdims=True)
        bidx = jnp.min(jnp.where(um == bm, lane, BIG_I32), axis=1,
                       keepdims=True)

        @pl.when(j == 0)
        def _():
            m_sc[...] = bm
            s_sc[...] = jnp.sum(jnp.exp(um - bm), axis=1, keepdims=True)
            i_sc[...] = bidx

        @pl.when(j > 0)
        def _():
            m_old = m_sc[...]
            m_new = jnp.maximum(m_old, bm)
            s_sc[...] = (s_sc[...] * jnp.exp(m_old - m_new)
                         + jnp.sum(jnp.exp(um - m_new), axis=1,
                                   keepdims=True))
            i_sc[...] = jnp.where(bm > m_old, bidx, i_sc[...])
            m_sc[...] = m_new

    @pl.when(ph == 0)
    def _phase0():
        qs = qs_ref[...]
        u = EXPLOIT * (qm_ref[...] + qs * eps_ref[...]) + (1.0 - EXPLOIT) * qs
        u_sc[:, pl.ds(off, NB)] = u
        lane = lax.broadcasted_iota(jnp.int32, (B, NB), 1) + off

        @pl.when(j < NBLK - 1)
        def _():
            _update(u, u, lane)

        @pl.when(j == NBLK - 1)
        def _():
            um = jnp.where(lane < N, u, NEG_INF)
            _update(u, um, lane)
            m_out_ref[...] = m_sc[...]
            pltpu.make_async_copy(i_sc, idx_smem, gsem).start()
            pltpu.make_async_copy(i_sc, idx_smem, gsem).wait()
            for b in range(B):
                col0 = jnp.minimum((idx_smem[b, 0] // 128) * 128, N - 128)
                pltpu.make_async_copy(
                    at_ref.at[b, :, pl.ds(pl.multiple_of(col0, 128), 128)],
                    win.at[b], sem).start()

    @pl.when(ph == 1)
    def _phase1():
        lse = m_sc[...] + jnp.log(s_sc[...])
        lp_ref[...] = u_sc[:, pl.ds(off, NB)] - lse

        @pl.when(j == NBLK - 1)
        def _gather():
            for b in range(B):
                col0 = jnp.minimum((idx_smem[b, 0] // 128) * 128, N - 128)
                pltpu.make_async_copy(
                    at_ref.at[b, :, pl.ds(pl.multiple_of(col0, 128), 128)],
                    win.at[b], sem).wait()
            idxv = i_sc[...].reshape(B, 1, 1)
            cw = idxv - jnp.minimum((idxv // 128) * 128, N - 128)
            lane3 = lax.broadcasted_iota(jnp.int32, (B, A, 128), 2)
            ba_ref[...] = jnp.sum(
                jnp.where(lane3 == cw, win[...], 0.0), axis=2)


def _tc_call(q_mean, q_std, eps, at):
    in_spec = pl.BlockSpec((B, NB), lambda ph, j: (0, jnp.where(ph == 0, j, 0)))
    return pl.pallas_call(
        _tc_body,
        grid=(2, NBLK),
        in_specs=[
            in_spec, in_spec, in_spec,
            pl.BlockSpec(memory_space=pl.ANY),
        ],
        out_specs=[
            pl.BlockSpec((B, NB), lambda ph, j: (0, jnp.where(ph == 0, 0, j))),
            pl.BlockSpec((B, 1), lambda ph, j: (0, 0)),
            pl.BlockSpec((B, A), lambda ph, j: (0, 0)),
        ],
        out_shape=[
            jax.ShapeDtypeStruct((B, N), jnp.float32),
            jax.ShapeDtypeStruct((B, 1), jnp.float32),
            jax.ShapeDtypeStruct((B, A), jnp.float32),
        ],
        scratch_shapes=[
            pltpu.VMEM((B, NPAD), jnp.float32),
            pltpu.VMEM((B, 1), jnp.float32),
            pltpu.VMEM((B, 1), jnp.float32),
            pltpu.VMEM((B, 1), jnp.int32),
            pltpu.SMEM((B, 1), jnp.int32),
            pltpu.VMEM((B, A, 128), jnp.float32),
            pltpu.SemaphoreType.DMA,
            pltpu.SemaphoreType.DMA,
        ],
        compiler_params=pltpu.CompilerParams(
            dimension_semantics=("arbitrary", "arbitrary")),
    )(q_mean, q_std, eps, at)


def kernel(q_mean, q_std, eps, action):
    at = action.transpose(0, 2, 1)
    log_probs, m, best_action = _tc_call(q_mean, q_std, eps, at)
    return log_probs, best_action, m.reshape(B)


# speedup vs baseline: 51.9228x; 1.0759x over previous
import jax
import jax.numpy as jnp
from jax import lax
from jax.experimental import pallas as pl
from jax.experimental.pallas import tpu as pltpu

B = 32
N = 100000
A = 8
NB = 25600
NBLK = (N + NB - 1) // NB
NPAD = NBLK * NB
EXPLOIT = 0.9
NEG_INF = float("-inf")
BIG_I32 = 2**30


def _tc_body(qm_ref, qs_ref, eps_ref, at_ref,
             lp_ref, m_out_ref, ba_ref,
             u_sc, m_sc, s_sc, i_sc, idx_smem, win, sem, gsem):
    ph = pl.program_id(0)
    j = pl.program_id(1)
    off = pl.multiple_of(j * NB, NB)

    def _update(u, um, lane):
        bm = jnp.max(um, axis=1, keepdims=True)
        bidx = jnp.min(jnp.where(um == bm, lane, BIG_I32), axis=1,
                       keepdims=True)

        @pl.when(j == 0)
        def _():
            m_sc[...] = bm
            s_sc[...] = jnp.sum(jnp.exp(um - bm), axis=1, keepdims=True)
            i_sc[...] = bidx

        @pl.when(j > 0)
        def _():
            m_old = m_sc[...]
            m_new = jnp.maximum(m_old, bm)
            s_sc[...] = (s_sc[...] * jnp.exp(m_old - m_new)
                         + jnp.sum(jnp.exp(um - m_new), axis=1,
                                   keepdims=True))
            i_sc[...] = jnp.where(bm > m_old, bidx, i_sc[...])
            m_sc[...] = m_new

    @pl.when(ph == 0)
    def _phase0():
        qs = qs_ref[...]
        u = EXPLOIT * (qm_ref[...] + qs * eps_ref[...]) + (1.0 - EXPLOIT) * qs
        u_sc[:, pl.ds(off, NB)] = u
        lane = lax.broadcasted_iota(jnp.int32, (B, NB), 1) + off

        @pl.when(j < NBLK - 1)
        def _():
            _update(u, u, lane)

        @pl.when(j == NBLK - 1)
        def _():
            um = jnp.where(lane < N, u, NEG_INF)
            _update(u, um, lane)
            m_out_ref[...] = m_sc[...]
            pltpu.make_async_copy(i_sc, idx_smem, gsem).start()
            pltpu.make_async_copy(i_sc, idx_smem, gsem).wait()
            for b in range(B):
                col0 = jnp.minimum((idx_smem[b, 0] // 128) * 128, N - 128)
                pltpu.make_async_copy(
                    at_ref.at[b, :, pl.ds(pl.multiple_of(col0, 128), 128)],
                    win.at[b], sem).start()

    @pl.when(ph == 1)
    def _phase1():
        lse = m_sc[...] + jnp.log(s_sc[...])
        lp_ref[...] = u_sc[:, pl.ds(off, NB)] - lse

        @pl.when(j == NBLK - 1)
        def _gather():
            for b in range(B):
                col0 = jnp.minimum((idx_smem[b, 0] // 128) * 128, N - 128)
                pltpu.make_async_copy(
                    at_ref.at[b, :, pl.ds(pl.multiple_of(col0, 128), 128)],
                    win.at[b], sem).wait()
            idxv = i_sc[...].reshape(B, 1, 1)
            cw = idxv - jnp.minimum((idxv // 128) * 128, N - 128)
            lane3 = lax.broadcasted_iota(jnp.int32, (B, A, 128), 2)
            ba_ref[...] = jnp.sum(
                jnp.where(lane3 == cw, win[...], 0.0), axis=2)


def _tc_call(q_mean, q_std, eps, at):
    in_spec = pl.BlockSpec((B, NB), lambda ph, j: (0, jnp.where(ph == 0, j, 0)))
    return pl.pallas_call(
        _tc_body,
        grid=(2, NBLK),
        in_specs=[
            in_spec, in_spec, in_spec,
            pl.BlockSpec(memory_space=pl.ANY),
        ],
        out_specs=[
            pl.BlockSpec((B, NB), lambda ph, j: (0, jnp.where(ph == 0, 0, j))),
            pl.BlockSpec((B, 1), lambda ph, j: (0, 0)),
            pl.BlockSpec((B, A), lambda ph, j: (0, 0)),
        ],
        out_shape=[
            jax.ShapeDtypeStruct((B, N), jnp.float32),
            jax.ShapeDtypeStruct((B, 1), jnp.float32),
            jax.ShapeDtypeStruct((B, A), jnp.float32),
        ],
        scratch_shapes=[
            pltpu.VMEM((B, NPAD), jnp.float32),
            pltpu.VMEM((B, 1), jnp.float32),
            pltpu.VMEM((B, 1), jnp.float32),
            pltpu.VMEM((B, 1), jnp.int32),
            pltpu.SMEM((B, 1), jnp.int32),
            pltpu.VMEM((B, A, 128), jnp.float32),
            pltpu.SemaphoreType.DMA,
            pltpu.SemaphoreType.DMA,
        ],
        compiler_params=pltpu.CompilerParams(
            dimension_semantics=("arbitrary", "arbitrary")),
    )(q_mean, q_std, eps, at)


def kernel(q_mean, q_std, eps, action):
    at = action.transpose(0, 2, 1)
    log_probs, m, best_action = _tc_call(q_mean, q_std, eps, at)
    return log_probs, best_action, m.reshape(B)
